# Initial kernel scaffold; baseline (speedup 1.0000x reference)
import jax
import jax.numpy as jnp
from jax.experimental import pallas as pl


def kernel(grad_output, input_ids, hidden_states_fp32, rstd, norm_weight):
    raise NotImplementedError("write your pallas kernel here")


# speedup vs baseline: 1.4174x; 1.4174x over previous
import jax
import jax.numpy as jnp
from jax.experimental import pallas as pl
from jax.experimental.pallas import tpu as pltpu

VOCAB = 65536
H = 4096
N_TOK = 4096
TOK_BLK = 512


def _stage_a_body(go_ref, hs_ref, rstd_ref, nw_ref, gh_ref, gnw_ref, acc_ref):
    i = pl.program_id(0)
    go = go_ref[...]
    r = rstd_ref[...]
    nw = nw_ref[...]
    gh_ref[...] = (go * (nw[None, :] * r)).astype(jnp.bfloat16)
    partial = jnp.sum(go * hs_ref[...] * r, axis=0, keepdims=True)

    @pl.when(i == 0)
    def _init():
        acc_ref[...] = partial

    @pl.when(i > 0)
    def _acc():
        acc_ref[...] += partial

    @pl.when(i == pl.num_programs(0) - 1)
    def _fin():
        gnw_ref[...] = acc_ref[...].astype(jnp.bfloat16)


def _stage_a(go, hs, rstd, nw):
    grid = N_TOK // TOK_BLK
    return pl.pallas_call(
        _stage_a_body,
        grid=(grid,),
        in_specs=[
            pl.BlockSpec((TOK_BLK, H), lambda i: (i, 0)),
            pl.BlockSpec((TOK_BLK, H), lambda i: (i, 0)),
            pl.BlockSpec((TOK_BLK, 1), lambda i: (i, 0)),
            pl.BlockSpec((H,), lambda i: (0,)),
        ],
        out_specs=[
            pl.BlockSpec((TOK_BLK, H), lambda i: (i, 0)),
            pl.BlockSpec((1, H), lambda i: (0, 0)),
        ],
        out_shape=[
            jax.ShapeDtypeStruct((N_TOK, H), jnp.bfloat16),
            jax.ShapeDtypeStruct((1, H), jnp.bfloat16),
        ],
        scratch_shapes=[pltpu.VMEM((1, H), jnp.float32)],
    )(go, hs, rstd, nw)


def kernel(grad_output, input_ids, hidden_states_fp32, rstd, norm_weight):
    go = grad_output.reshape(N_TOK, H)
    hs = hidden_states_fp32.reshape(N_TOK, H)
    r = rstd.reshape(N_TOK, 1)
    ids = input_ids.reshape(N_TOK).astype(jnp.int32)
    gh, gnw = _stage_a(go, hs, r, norm_weight)
    table = jnp.zeros((VOCAB, H), jnp.bfloat16).at[ids].add(gh)
    return (table, gnw.reshape(H))


# speedup vs baseline: 2.1338x; 1.5055x over previous
import functools

import jax
import jax.numpy as jnp
from jax import lax
from jax.experimental import pallas as pl
from jax.experimental.pallas import tpu as pltpu
from jax.experimental.pallas import tpu_sc as plsc

VOCAB = 65536
H = 4096
N_TOK = 4096
TOK_BLK = 256

NC, NS = 2, 16
NW = NC * NS
STRIPE = VOCAB // NW
NGRP = STRIPE // 8
NEG = -(2**31) + 1


def _stage_a_body(go_ref, hs_ref, rstd_ref, nw_ref, par_ref,
                  ghp_ref, gnw_ref, acc_ref):
    i = pl.program_id(0)
    go = go_ref[...]
    r = rstd_ref[...]
    nw = nw_ref[...]
    par = par_ref[...]
    gh = (go * (nw[None, :] * r)).astype(jnp.bfloat16)
    even = gh * (1.0 - par).astype(jnp.bfloat16)
    odd = gh * par.astype(jnp.bfloat16)
    pair = jnp.stack([even, odd], axis=1)
    ghp_ref[...] = pair.reshape(2 * TOK_BLK, H)
    partial = jnp.sum(go * hs_ref[...] * r, axis=0, keepdims=True)

    @pl.when(i == 0)
    def _init():
        acc_ref[...] = partial

    @pl.when(i > 0)
    def _acc():
        acc_ref[...] += partial

    @pl.when(i == pl.num_programs(0) - 1)
    def _fin():
        gnw_ref[...] = acc_ref[...].astype(jnp.bfloat16)


def _stage_a(go, hs, rstd, nw, par):
    grid = N_TOK // TOK_BLK
    return pl.pallas_call(
        _stage_a_body,
        grid=(grid,),
        in_specs=[
            pl.BlockSpec((TOK_BLK, H), lambda i: (i, 0)),
            pl.BlockSpec((TOK_BLK, H), lambda i: (i, 0)),
            pl.BlockSpec((TOK_BLK, 1), lambda i: (i, 0)),
            pl.BlockSpec((H,), lambda i: (0,)),
            pl.BlockSpec((TOK_BLK, 1), lambda i: (i, 0)),
        ],
        out_specs=[
            pl.BlockSpec((2 * TOK_BLK, H), lambda i: (i, 0)),
            pl.BlockSpec((1, H), lambda i: (0, 0)),
        ],
        out_shape=[
            jax.ShapeDtypeStruct((2 * N_TOK, H), jnp.bfloat16),
            jax.ShapeDtypeStruct((1, H), jnp.bfloat16),
        ],
        scratch_shapes=[pltpu.VMEM((1, H), jnp.float32)],
    )(go, hs, rstd, nw, par)


def _scalar(x):
    return jnp.max(x) if x.ndim else x


def _extract(ref, j, fill):
    b = (j // 16) * 16
    vec = ref[pl.ds(b, 16)]
    m = lax.iota(jnp.int32, 16) == (j - b)
    return jnp.max(jnp.where(m, vec, fill))


def _sc_body(ids_hbm, ghp_hbm, zrows_hbm, out_hbm,
             ids_v, tl, rl, m2, m3, zb, grpbuf, stgrp, touched, semz):
    c = lax.axis_index("c")
    s = lax.axis_index("s")
    wid = s * NC + c
    base = wid * STRIPE

    pltpu.sync_copy(ids_hbm, ids_v)
    pltpu.sync_copy(zrows_hbm, zb)
    pltpu.sync_copy(zrows_hbm, grpbuf)

    iota16 = lax.iota(jnp.int32, 16)

    def _clear(i, carry):
        touched[pl.ds(i * 16, 16)] = jnp.zeros((16,), jnp.int32)
        return carry

    lax.fori_loop(0, NGRP // 16, _clear, 0)

    def _route(i, cnt):
        v = ids_v[pl.ds(i * 16, 16)]
        m = (v >= base) & (v < base + STRIPE)
        plsc.store_compressed(tl.at[pl.ds(cnt, 16)], iota16 + i * 16, mask=m)
        plsc.store_compressed(rl.at[pl.ds(cnt, 16)], v - base, mask=m)
        return cnt + _scalar(plsc.all_reduce_population_count(m))

    cnt = lax.fori_loop(0, N_TOK // 16, _route, jnp.int32(0))

    def _mark(k, carry):
        g = _extract(rl, k, jnp.int32(0)) // 8
        plsc.store_scatter(touched, [jnp.broadcast_to(g, (16,))],
                           jnp.ones((16,), jnp.int32), mask=iota16 == 0)
        return carry

    lax.fori_loop(0, cnt, _mark, 0)

    def _fire(gi, carry):
        f = _extract(touched, gi, jnp.int32(0))

        @pl.when(f == 0)
        def _():
            pltpu.make_async_copy(
                zb, out_hbm.at[pl.ds(base + gi * 8, 8)], semz).start()

        return carry

    lax.fori_loop(0, NGRP, _fire, 0)

    def _cond(j):
        return j < cnt

    def _step(j):
        r_j = _extract(rl, j, jnp.int32(NEG))

        @pl.when(r_j >= 0)
        def _process():
            g = r_j // 8
            nv = (cnt + 15) // 16

            def _collect(w, mcnt):
                b = w * 16
                rv = rl[pl.ds(b, 16)]
                eq = (rv // 8 == g) & ((iota16 + b) < cnt)
                plsc.store_compressed(m2.at[pl.ds(mcnt, 16)],
                                      tl[pl.ds(b, 16)], mask=eq)
                plsc.store_compressed(m3.at[pl.ds(mcnt, 16)], rv, mask=eq)
                rl[pl.ds(b, 16)] = jnp.where(eq, jnp.int32(-1), rv)
                return mcnt + _scalar(plsc.all_reduce_population_count(eq))

            mcnt = lax.fori_loop(0, nv, _collect, jnp.int32(0))

            def _addk(k, carry):
                tk = _extract(m2, k, jnp.int32(0))
                rk = _extract(m3, k, jnp.int32(0))
                q = pl.multiple_of(2 * ((rk % 8) // 2), 2)
                qs = pl.multiple_of(2 * (tk % 4), 2)
                pltpu.sync_copy(ghp_hbm.at[pl.ds((tk // 4) * 8, 8)], stgrp)

                def _vadd(cc, c2):
                    co = pl.multiple_of(cc * 16, 16)
                    grpbuf[pl.ds(q, 2), pl.ds(co, 16)] = (
                        grpbuf[pl.ds(q, 2), pl.ds(co, 16)]
                        + stgrp[pl.ds(qs, 2), pl.ds(co, 16)])
                    return c2

                lax.fori_loop(0, H // 16, _vadd, 0)
                return carry

            lax.fori_loop(0, mcnt, _addk, 0)
            pltpu.sync_copy(grpbuf, out_hbm.at[pl.ds(base + g * 8, 8)])

            def _rez(k, carry):
                rk = _extract(m3, k, jnp.int32(0))
                q = pl.multiple_of(2 * ((rk % 8) // 2), 2)

                def _vz(cc, c2):
                    co = pl.multiple_of(cc * 16, 16)
                    grpbuf[pl.ds(q, 2), pl.ds(co, 16)] = jnp.zeros(
                        (2, 16), jnp.bfloat16)
                    return c2

                lax.fori_loop(0, H // 16, _vz, 0)
                return carry

            lax.fori_loop(0, mcnt, _rez, 0)

        return j + 1

    lax.while_loop(_cond, _step, jnp.int32(0))

    def _drain(gi, carry):
        f = _extract(touched, gi, jnp.int32(0))

        @pl.when(f == 0)
        def _():
            pltpu.make_async_copy(
                zb, out_hbm.at[pl.ds(base + gi * 8, 8)], semz).wait()

        return carry

    lax.fori_loop(0, NGRP, _drain, 0)


_sc_scatter = functools.partial(
    pl.kernel,
    out_type=jax.ShapeDtypeStruct((VOCAB, H), jnp.bfloat16),
    mesh=plsc.VectorSubcoreMesh(core_axis_name="c", subcore_axis_name="s"),
    compiler_params=pltpu.CompilerParams(needs_layout_passes=False),
    scratch_types=[
        pltpu.VMEM((N_TOK,), jnp.int32),
        pltpu.VMEM((N_TOK + 16,), jnp.int32),
        pltpu.VMEM((N_TOK + 16,), jnp.int32),
        pltpu.VMEM((N_TOK + 16,), jnp.int32),
        pltpu.VMEM((N_TOK + 16,), jnp.int32),
        pltpu.VMEM((8, H), jnp.bfloat16),
        pltpu.VMEM((8, H), jnp.bfloat16),
        pltpu.VMEM((8, H), jnp.bfloat16),
        pltpu.VMEM((NGRP,), jnp.int32),
        pltpu.SemaphoreType.DMA,
    ],
)(_sc_body)


def kernel(grad_output, input_ids, hidden_states_fp32, rstd, norm_weight):
    go = grad_output.reshape(N_TOK, H)
    hs = hidden_states_fp32.reshape(N_TOK, H)
    r = rstd.reshape(N_TOK, 1)
    ids = input_ids.reshape(N_TOK).astype(jnp.int32)
    par = (ids % 2).astype(jnp.float32).reshape(N_TOK, 1)
    ghp, gnw = _stage_a(go, hs, r, norm_weight, par)
    zrows = jnp.zeros((8, H), jnp.bfloat16)
    table = _sc_scatter(ids, ghp, zrows)
    return (table, gnw.reshape(H))


# speedup vs baseline: 2.6680x; 1.2503x over previous
import functools

import jax
import jax.numpy as jnp
from jax import lax
from jax.experimental import pallas as pl
from jax.experimental.pallas import tpu as pltpu
from jax.experimental.pallas import tpu_sc as plsc

VOCAB = 65536
H = 4096
N_TOK = 4096
TOK_BLK = 256

NC, NS = 2, 16
NW = NC * NS
STRIPE = VOCAB // NW
NGRP = STRIPE // 8
NEG = -(2**31) + 1


def _stage_a_body(go_ref, hs_ref, rstd_ref, nw_ref, par_ref,
                  ghp_ref, gnw_ref, acc_ref):
    i = pl.program_id(0)
    go = go_ref[...]
    r = rstd_ref[...]
    nw = nw_ref[...]
    par = par_ref[...]
    gh = (go * (nw[None, :] * r)).astype(jnp.bfloat16)
    even = gh * (1.0 - par).astype(jnp.bfloat16)
    odd = gh * par.astype(jnp.bfloat16)
    pair = jnp.stack([even, odd], axis=1)
    ghp_ref[...] = pair.reshape(2 * TOK_BLK, H)
    partial = jnp.sum(go * hs_ref[...] * r, axis=0, keepdims=True)

    @pl.when(i == 0)
    def _init():
        acc_ref[...] = partial

    @pl.when(i > 0)
    def _acc():
        acc_ref[...] += partial

    @pl.when(i == pl.num_programs(0) - 1)
    def _fin():
        gnw_ref[...] = acc_ref[...].astype(jnp.bfloat16)


def _stage_a(go, hs, rstd, nw, par):
    grid = N_TOK // TOK_BLK
    return pl.pallas_call(
        _stage_a_body,
        grid=(grid,),
        in_specs=[
            pl.BlockSpec((TOK_BLK, H), lambda i: (i, 0)),
            pl.BlockSpec((TOK_BLK, H), lambda i: (i, 0)),
            pl.BlockSpec((TOK_BLK, 1), lambda i: (i, 0)),
            pl.BlockSpec((H,), lambda i: (0,)),
            pl.BlockSpec((TOK_BLK, 1), lambda i: (i, 0)),
        ],
        out_specs=[
            pl.BlockSpec((2 * TOK_BLK, H), lambda i: (i, 0)),
            pl.BlockSpec((1, H), lambda i: (0, 0)),
        ],
        out_shape=[
            jax.ShapeDtypeStruct((2 * N_TOK, H), jnp.bfloat16),
            jax.ShapeDtypeStruct((1, H), jnp.bfloat16),
        ],
        scratch_shapes=[pltpu.VMEM((1, H), jnp.float32)],
    )(go, hs, rstd, nw, par)


def _scalar(x):
    return jnp.max(x) if x.ndim else x


def _extract(ref, j, fill):
    b = (j // 16) * 16
    vec = ref[pl.ds(b, 16)]
    m = lax.iota(jnp.int32, 16) == (j - b)
    return jnp.max(jnp.where(m, vec, fill))


def _sc_body(ids_hbm, ghp_hbm, zrows_hbm, out_hbm,
             ids_v, tl, rl, m2, m3, zb, grpbuf, stgrp, touched, zsh,
             semz, semr0, semr1, semg0, semg1):
    c = lax.axis_index("c")
    s = lax.axis_index("s")
    wid = s * NC + c
    base = wid * STRIPE

    pltpu.sync_copy(ids_hbm, ids_v)
    pltpu.sync_copy(zrows_hbm, zb)

    @pl.when(s == 0)
    def _():
        pltpu.sync_copy(zrows_hbm, zsh)

    plsc.subcore_barrier()

    iota16 = lax.iota(jnp.int32, 16)

    def _clear(i, carry):
        touched[pl.ds(i * 16, 16)] = jnp.zeros((16,), jnp.int32)
        return carry

    lax.fori_loop(0, NGRP // 16, _clear, 0)

    def _route(i, cnt):
        v = ids_v[pl.ds(i * 16, 16)]
        m = (v >= base) & (v < base + STRIPE)
        plsc.store_compressed(tl.at[pl.ds(cnt, 16)], iota16 + i * 16, mask=m)
        plsc.store_compressed(rl.at[pl.ds(cnt, 16)], v - base, mask=m)
        return cnt + _scalar(plsc.all_reduce_population_count(m))

    cnt = lax.fori_loop(0, N_TOK // 16, _route, jnp.int32(0))

    def _mark(k, carry):
        g = _extract(rl, k, jnp.int32(0)) // 8
        plsc.store_scatter(touched, [jnp.broadcast_to(g, (16,))],
                           jnp.ones((16,), jnp.int32), mask=iota16 == 0)
        return carry

    lax.fori_loop(0, cnt, _mark, 0)

    def _fire(gi, carry):
        f = _extract(touched, gi, jnp.int32(0))

        @pl.when(f == 0)
        def _():
            pltpu.make_async_copy(
                zb, out_hbm.at[pl.ds(base + gi * 8, 8)], semz).start()

        return carry

    lax.fori_loop(0, NGRP, _fire, 0)

    pltpu.make_async_copy(zsh, grpbuf.at[0], semr0).start()
    pltpu.make_async_copy(zsh, grpbuf.at[1], semr1).start()

    def _cond(st):
        return st[0] < cnt

    def _step(st):
        j, gb = st
        r_j = _extract(rl, j, jnp.int32(NEG))

        def _process(gb):
            g = r_j // 8
            nv = (cnt + 15) // 16

            def _collect(w, mcnt):
                b = w * 16
                rv = rl[pl.ds(b, 16)]
                eq = (rv // 8 == g) & ((iota16 + b) < cnt)
                plsc.store_compressed(m2.at[pl.ds(mcnt, 16)],
                                      tl[pl.ds(b, 16)], mask=eq)
                plsc.store_compressed(m3.at[pl.ds(mcnt, 16)], rv, mask=eq)
                rl[pl.ds(b, 16)] = jnp.where(eq, jnp.int32(-1), rv)
                return mcnt + _scalar(plsc.all_reduce_population_count(eq))

            mcnt = lax.fori_loop(0, nv, _collect, jnp.int32(0))

            @pl.when(gb == 0)
            def _():
                pltpu.make_async_copy(zsh, grpbuf.at[0], semr0).wait()

            @pl.when(gb == 1)
            def _():
                pltpu.make_async_copy(zsh, grpbuf.at[1], semr1).wait()

            def _issue(k, sb):
                tk = _extract(m2, k, jnp.int32(0))

                @pl.when(sb == 0)
                def _():
                    pltpu.make_async_copy(
                        ghp_hbm.at[pl.ds((tk // 4) * 8, 8)],
                        stgrp.at[0], semg0).start()

                @pl.when(sb == 1)
                def _():
                    pltpu.make_async_copy(
                        ghp_hbm.at[pl.ds((tk // 4) * 8, 8)],
                        stgrp.at[1], semg1).start()

            def _wait(k, sb):
                tk = _extract(m2, k, jnp.int32(0))

                @pl.when(sb == 0)
                def _():
                    pltpu.make_async_copy(
                        ghp_hbm.at[pl.ds((tk // 4) * 8, 8)],
                        stgrp.at[0], semg0).wait()

                @pl.when(sb == 1)
                def _():
                    pltpu.make_async_copy(
                        ghp_hbm.at[pl.ds((tk // 4) * 8, 8)],
                        stgrp.at[1], semg1).wait()

            _issue(jnp.int32(0), jnp.int32(0))

            def _addk(k, carry):
                sb = k % 2
                tk = _extract(m2, k, jnp.int32(0))
                rk = _extract(m3, k, jnp.int32(0))
                _wait(k, sb)

                @pl.when(k + 1 < mcnt)
                def _():
                    _issue(k + 1, (k + 1) % 2)

                q = pl.multiple_of(2 * ((rk % 8) // 2), 2)
                qs = pl.multiple_of(2 * (tk % 4), 2)

                def _vadd(cc, c2):
                    for u in range(4):
                        co = pl.multiple_of(cc * 64 + u * 16, 16)
                        grpbuf[gb, pl.ds(q, 2), pl.ds(co, 16)] = (
                            grpbuf[gb, pl.ds(q, 2), pl.ds(co, 16)]
                            + stgrp[sb, pl.ds(qs, 2), pl.ds(co, 16)])
                    return c2

                lax.fori_loop(0, H // 64, _vadd, 0)
                return carry

            lax.fori_loop(0, mcnt, _addk, 0)
            pltpu.sync_copy(grpbuf.at[gb], out_hbm.at[pl.ds(base + g * 8, 8)])

            @pl.when(gb == 0)
            def _():
                pltpu.make_async_copy(zsh, grpbuf.at[0], semr0).start()

            @pl.when(gb == 1)
            def _():
                pltpu.make_async_copy(zsh, grpbuf.at[1], semr1).start()

            return 1 - gb

        gb = lax.cond(r_j >= 0, _process, lambda b: b, gb)
        return (j + 1, gb)

    lax.while_loop(_cond, _step, (jnp.int32(0), jnp.int32(0)))

    pltpu.make_async_copy(zsh, grpbuf.at[0], semr0).wait()
    pltpu.make_async_copy(zsh, grpbuf.at[1], semr1).wait()

    def _drain(gi, carry):
        f = _extract(touched, gi, jnp.int32(0))

        @pl.when(f == 0)
        def _():
            pltpu.make_async_copy(
                zb, out_hbm.at[pl.ds(base + gi * 8, 8)], semz).wait()

        return carry

    lax.fori_loop(0, NGRP, _drain, 0)


_sc_scatter = functools.partial(
    pl.kernel,
    out_type=jax.ShapeDtypeStruct((VOCAB, H), jnp.bfloat16),
    mesh=plsc.VectorSubcoreMesh(core_axis_name="c", subcore_axis_name="s"),
    compiler_params=pltpu.CompilerParams(needs_layout_passes=False),
    scratch_types=[
        pltpu.VMEM((N_TOK,), jnp.int32),
        pltpu.VMEM((N_TOK + 16,), jnp.int32),
        pltpu.VMEM((N_TOK + 16,), jnp.int32),
        pltpu.VMEM((N_TOK + 16,), jnp.int32),
        pltpu.VMEM((N_TOK + 16,), jnp.int32),
        pltpu.VMEM((8, H), jnp.bfloat16),
        pltpu.VMEM((2, 8, H), jnp.bfloat16),
        pltpu.VMEM((2, 8, H), jnp.bfloat16),
        pltpu.VMEM((NGRP,), jnp.int32),
        pltpu.VMEM_SHARED((8, H), jnp.bfloat16),
        pltpu.SemaphoreType.DMA,
        pltpu.SemaphoreType.DMA,
        pltpu.SemaphoreType.DMA,
        pltpu.SemaphoreType.DMA,
        pltpu.SemaphoreType.DMA,
    ],
)(_sc_body)


def kernel(grad_output, input_ids, hidden_states_fp32, rstd, norm_weight):
    go = grad_output.reshape(N_TOK, H)
    hs = hidden_states_fp32.reshape(N_TOK, H)
    r = rstd.reshape(N_TOK, 1)
    ids = input_ids.reshape(N_TOK).astype(jnp.int32)
    par = (ids % 2).astype(jnp.float32).reshape(N_TOK, 1)
    ghp, gnw = _stage_a(go, hs, r, norm_weight, par)
    zrows = jnp.zeros((8, H), jnp.bfloat16)
    table = _sc_scatter(ids, ghp, zrows)
    return (table, gnw.reshape(H))
